# Initial kernel scaffold; baseline (speedup 1.0000x reference)
import jax
import jax.numpy as jnp
from jax.experimental import pallas as pl


def kernel(V, global_state, graph_size, W):
    raise NotImplementedError("write your pallas kernel here")


# speedup vs baseline: 2.7808x; 2.7808x over previous
import functools

import jax
import jax.numpy as jnp
from jax.experimental import pallas as pl
from jax.experimental.pallas import tpu as pltpu

_BN = 1024


def _body(graph_size_ref, global_state_ref, W_ref, V_ref, out_ref, gs_ref):
    b = pl.program_id(0)
    nb = pl.program_id(1)
    gs_all = jnp.dot(global_state_ref[...], W_ref[...],
                     preferred_element_type=jnp.float32)
    gs_ref[...] = gs_all
    bid = jax.lax.broadcasted_iota(jnp.int32, gs_all.shape, 0)
    gs_row = jnp.sum(jnp.where(bid == b, gs_all, 0.0), axis=0, keepdims=True)
    gsize = graph_size_ref[b]
    bn, o = _BN, gs_all.shape[1]
    row = jax.lax.broadcasted_iota(jnp.int32, (bn, o), 0) + nb * bn
    tail = jnp.where(row < gsize, jnp.broadcast_to(gs_row, (bn, o)), 0.0)
    out_ref[...] = jnp.concatenate([V_ref[0], tail], axis=-1)[None]


@functools.partial(jax.jit, static_argnames=("interpret",))
def kernel(V, global_state, graph_size, W, interpret=False):
    b, N, Ov = V.shape
    O = W.shape[1]
    grid = (b, N // _BN)
    out, gs = pl.pallas_call(
        _body,
        grid=grid,
        in_specs=[
            pl.BlockSpec(memory_space=pltpu.SMEM),
            pl.BlockSpec((b, global_state.shape[1]), lambda i, j: (0, 0)),
            pl.BlockSpec((Ov, O), lambda i, j: (0, 0)),
            pl.BlockSpec((1, _BN, Ov), lambda i, j: (i, j, 0)),
        ],
        out_specs=[
            pl.BlockSpec((1, _BN, Ov + O), lambda i, j: (i, j, 0)),
            pl.BlockSpec((b, O), lambda i, j: (0, 0)),
        ],
        out_shape=[
            jax.ShapeDtypeStruct((b, N, Ov + O), jnp.float32),
            jax.ShapeDtypeStruct((b, O), jnp.float32),
        ],
        interpret=interpret,
    )(graph_size, global_state, W, V)
    return out, gs
